# baseline (speedup 1.0000x reference)
import dataclasses
import functools

import jax
import jax.numpy as jnp
from jax import lax
from jax.experimental import pallas as pl
from jax.experimental.pallas import tpu as pltpu
from jax.experimental.pallas import tpu_sc as plsc

NC = 2
NS = 16
LANES = 16
CHUNK = 96


def _row_block(n):
    for blk in (2000, 1000, 500, 200, 100, 50, 25, 10, 8):
        if n % blk == 0:
            return blk
    return n


def _gates(x, W1, W2):
    n, d = x.shape
    blk = _row_block(n)

    def body(x_ref, w1_ref, w2_ref, o1_ref, o2_ref):
        xb = x_ref[...]
        o1_ref[...] = jnp.sum(xb * w1_ref[...], axis=1, keepdims=True)
        o2_ref[...] = jnp.sum(xb * w2_ref[...], axis=1, keepdims=True)

    o1, o2 = pl.pallas_call(
        body,
        grid=(n // blk,),
        in_specs=[
            pl.BlockSpec((blk, d), lambda i: (i, 0)),
            pl.BlockSpec((1, d), lambda i: (0, 0)),
            pl.BlockSpec((1, d), lambda i: (0, 0)),
        ],
        out_specs=[
            pl.BlockSpec((blk, 1), lambda i: (i, 0)),
            pl.BlockSpec((blk, 1), lambda i: (i, 0)),
        ],
        out_shape=[
            jax.ShapeDtypeStruct((n, 1), jnp.float32),
            jax.ShapeDtypeStruct((n, 1), jnp.float32),
        ],
    )(x, W1, W2)
    return o1.reshape(n), o2.reshape(n)


def _sum_partials(p):
    _, n, d = p.shape
    blk = _row_block(n)

    def body(p_ref, o_ref):
        o_ref[...] = p_ref[0] + p_ref[1]

    return pl.pallas_call(
        body,
        grid=(n // blk,),
        in_specs=[pl.BlockSpec((2, blk, d), lambda i: (0, i, 0))],
        out_specs=pl.BlockSpec((blk, d), lambda i: (i, 0)),
        out_shape=jax.ShapeDtypeStruct((n, d), jnp.float32),
    )(p)


def _sc_aggregate(x, src, dst, adj, x1, x2):
    n, d = x.shape
    epad = src.shape[0]
    epw = epad // (NC * NS)
    nchunks = epw // CHUNK
    assert nchunks % 2 == 0
    zrows = (n // NS) // 8 * 8
    tail = n - zrows * NS
    zsizes = []
    left = zrows
    while left > 0:
        blk = min(left, CHUNK)
        zsizes.append(blk)
        left -= blk

    mesh = plsc.VectorSubcoreMesh(core_axis_name="c", subcore_axis_name="s")
    cp = pltpu.CompilerParams()
    if "needs_layout_passes" in pltpu.CompilerParams.__dataclass_fields__:
        cp = dataclasses.replace(cp, needs_layout_passes=False)

    @functools.partial(
        pl.kernel,
        out_type=jax.ShapeDtypeStruct((NC, n, d), jnp.float32),
        mesh=mesh,
        compiler_params=cp,
        scratch_types=[
            pltpu.VMEM((n,), jnp.float32),
            pltpu.VMEM((n,), jnp.float32),
            pltpu.VMEM((CHUNK,), jnp.int32),
            pltpu.VMEM((CHUNK,), jnp.int32),
            pltpu.VMEM((CHUNK,), jnp.int32),
            pltpu.VMEM((CHUNK,), jnp.int32),
            pltpu.VMEM((CHUNK,), jnp.float32),
            pltpu.VMEM((CHUNK,), jnp.float32),
            pltpu.VMEM((CHUNK, d), jnp.float32),
            pltpu.VMEM((CHUNK, d), jnp.float32),
            pltpu.VMEM_SHARED((n, d), jnp.float32),
            pltpu.SemaphoreType.DMA,
            pltpu.SemaphoreType.DMA,
        ],
    )
    def sc_kernel(x_hbm, src_hbm, dst_hbm, adj_hbm, x1_hbm, x2_hbm, out_hbm,
                  x1t, x2t, ts0, ts1, td0, td1, ta0, ta1, rows0, rows1,
                  accum, gsem, ssem):
        c = lax.axis_index("c")
        s = lax.axis_index("s")
        bufs = ((ts0, td0, ta0, rows0), (ts1, td1, ta1, rows1))

        pltpu.sync_copy(x1_hbm, x1t)
        pltpu.sync_copy(x2_hbm, x2t)

        @pl.loop(0, CHUNK)
        def _zero_rows(i):
            for j in range(d // LANES):
                rows0[i, pl.ds(j * LANES, LANES)] = jnp.zeros((LANES,), jnp.float32)

        off = 0
        for blk in zsizes:
            pltpu.sync_copy(rows0.at[pl.ds(0, blk)],
                            accum.at[pl.ds(s * zrows + off, blk)])
            off += blk
        if tail:
            @pl.when(s == NS - 1)
            def _zero_tail():
                pltpu.sync_copy(rows0.at[pl.ds(0, tail)],
                                accum.at[pl.ds(NS * zrows, tail)])

        base0 = (c * NS + s) * epw

        pltpu.sync_copy(src_hbm.at[pl.ds(base0, CHUNK)], ts0)
        pltpu.sync_copy(dst_hbm.at[pl.ds(base0, CHUNK)], td0)
        pltpu.sync_copy(adj_hbm.at[pl.ds(base0, CHUNK)], ta0)
        pltpu.async_copy(x_hbm.at[td0], rows0, gsem)

        plsc.subcore_barrier()

        @pl.loop(0, nchunks, step=2)
        def _edge_chunks(k):
            for p in range(2):
                i = k + p
                tsrc, tdst, tadj, rows = bufs[p]
                tsrcq, tdstq, tadjq, rowsq = bufs[1 - p]

                pltpu.make_async_copy(x_hbm.at[tdst], rows, gsem).wait()

                @pl.when(i > 0)
                def _drain_scatter():
                    pltpu.make_async_copy(rowsq, accum.at[tsrcq], ssem).wait()

                @pl.when(i + 1 < nchunks)
                def _prefetch():
                    nbase = base0 + (i + 1) * CHUNK
                    pltpu.sync_copy(src_hbm.at[pl.ds(nbase, CHUNK)], tsrcq)
                    pltpu.sync_copy(dst_hbm.at[pl.ds(nbase, CHUNK)], tdstq)
                    pltpu.sync_copy(adj_hbm.at[pl.ds(nbase, CHUNK)], tadjq)
                    pltpu.async_copy(x_hbm.at[tdstq], rowsq, gsem)

                @pl.loop(0, CHUNK, step=LANES)
                def _gate_scale(g):
                    idxs = tsrc[pl.ds(g, LANES)]
                    idxd = tdst[pl.ds(g, LANES)]
                    s1 = plsc.load_gather(x1t, [idxs])
                    s2 = plsc.load_gather(x2t, [idxd])
                    e2 = jnp.exp((s1 + s2) * 2.0)
                    mv = (1.0 - 2.0 / (e2 + 1.0)) * tadj[pl.ds(g, LANES)]
                    for ii in range(LANES):
                        mi = mv[ii]
                        for j in range(d // LANES):
                            sl = pl.ds(j * LANES, LANES)
                            rows[g + ii, sl] = rows[g + ii, sl] * mi

                pltpu.async_copy(rows, accum.at[tsrc], ssem, add=True)

        pltpu.make_async_copy(rows1, accum.at[ts1], ssem).wait()

        plsc.subcore_barrier()

        r0 = s * zrows
        pltpu.sync_copy(accum.at[pl.ds(r0, zrows)], out_hbm.at[c, pl.ds(r0, zrows)])
        if tail:
            @pl.when(s == NS - 1)
            def _write_tail():
                pltpu.sync_copy(accum.at[pl.ds(NS * zrows, tail)],
                                out_hbm.at[c, pl.ds(NS * zrows, tail)])

    return sc_kernel(x, src, dst, adj, x1, x2)


def kernel(x, edge_index, adj_values, W1, W2):
    e = edge_index.shape[1]

    x1, x2 = _gates(x, W1, W2)

    quantum = NC * NS * CHUNK * 2
    epad = ((e + quantum - 1) // quantum) * quantum
    pad = epad - e
    src = jnp.concatenate([edge_index[0], jnp.zeros((pad,), jnp.int32)])
    dst = jnp.concatenate([edge_index[1], jnp.zeros((pad,), jnp.int32)])
    adj = jnp.concatenate([adj_values, jnp.zeros((pad,), jnp.float32)])

    partials = _sc_aggregate(x, src, dst, adj, x1, x2)
    return _sum_partials(partials)


# speedup vs baseline: 1.7449x; 1.7449x over previous
import dataclasses
import functools

import jax
import jax.numpy as jnp
from jax import lax
from jax.experimental import pallas as pl
from jax.experimental.pallas import tpu as pltpu
from jax.experimental.pallas import tpu_sc as plsc

NC = 2
NS = 16
LANES = 16
CHUNK = 112


def _row_block(n):
    for blk in (2000, 1000, 500, 200, 100, 50, 25, 10, 8):
        if n % blk == 0:
            return blk
    return n


def _gates(x, W1, W2):
    n, d = x.shape
    blk = _row_block(n)

    def body(x_ref, w1_ref, w2_ref, o1_ref, o2_ref):
        xb = x_ref[...]
        o1_ref[...] = jnp.sum(xb * w1_ref[...], axis=1, keepdims=True)
        o2_ref[...] = jnp.sum(xb * w2_ref[...], axis=1, keepdims=True)

    o1, o2 = pl.pallas_call(
        body,
        grid=(n // blk,),
        in_specs=[
            pl.BlockSpec((blk, d), lambda i: (i, 0)),
            pl.BlockSpec((1, d), lambda i: (0, 0)),
            pl.BlockSpec((1, d), lambda i: (0, 0)),
        ],
        out_specs=[
            pl.BlockSpec((blk, 1), lambda i: (i, 0)),
            pl.BlockSpec((blk, 1), lambda i: (i, 0)),
        ],
        out_shape=[
            jax.ShapeDtypeStruct((n, 1), jnp.float32),
            jax.ShapeDtypeStruct((n, 1), jnp.float32),
        ],
    )(x, W1, W2)
    return o1.reshape(n), o2.reshape(n)


def _sum_partials(p):
    _, n, d = p.shape
    blk = _row_block(n)

    def body(p_ref, o_ref):
        o_ref[...] = p_ref[0] + p_ref[1]

    return pl.pallas_call(
        body,
        grid=(n // blk,),
        in_specs=[pl.BlockSpec((2, blk, d), lambda i: (0, i, 0))],
        out_specs=pl.BlockSpec((blk, d), lambda i: (i, 0)),
        out_shape=jax.ShapeDtypeStruct((n, d), jnp.float32),
    )(p)


def _sc_aggregate(x, pk3, x1, x2):
    n, d = x.shape
    nctot = pk3.shape[0]
    cpw = nctot // (NC * NS)
    assert cpw % 6 == 0
    zrows = (n // NS) // 8 * 8
    tail = n - zrows * NS
    zsizes = []
    left = zrows
    while left > 0:
        blk = min(left, CHUNK)
        zsizes.append(blk)
        left -= blk

    mesh = plsc.VectorSubcoreMesh(core_axis_name="c", subcore_axis_name="s")
    cp = pltpu.CompilerParams()
    if "needs_layout_passes" in pltpu.CompilerParams.__dataclass_fields__:
        cp = dataclasses.replace(cp, needs_layout_passes=False)

    @functools.partial(
        pl.kernel,
        out_type=jax.ShapeDtypeStruct((NC, n, d), jnp.float32),
        mesh=mesh,
        compiler_params=cp,
        scratch_types=[
            pltpu.VMEM((n,), jnp.float32),
            pltpu.VMEM((n,), jnp.float32),
            pltpu.VMEM((3, CHUNK), jnp.int32),
            pltpu.VMEM((3, CHUNK), jnp.int32),
            pltpu.VMEM((3, CHUNK), jnp.int32),
            pltpu.VMEM((CHUNK, d), jnp.float32),
            pltpu.VMEM((CHUNK, d), jnp.float32),
            pltpu.VMEM_SHARED((n, d), jnp.float32),
            pltpu.SemaphoreType.DMA,
            pltpu.SemaphoreType.DMA,
            pltpu.SemaphoreType.DMA,
            pltpu.SemaphoreType.DMA,
        ],
    )
    def sc_kernel(x_hbm, pk3_hbm, x1_hbm, x2_hbm, out_hbm,
                  x1t, x2t, tb0, tb1, tb2, rows0, rows1,
                  accum, isem0, isem1, gsem, ssem):
        c = lax.axis_index("c")
        s = lax.axis_index("s")
        tbs = (tb0, tb1, tb2)
        rws = (rows0, rows1)
        isems = (isem0, isem1)

        pltpu.sync_copy(x1_hbm, x1t)
        pltpu.sync_copy(x2_hbm, x2t)

        @pl.loop(0, CHUNK)
        def _zero_rows(i):
            for j in range(d // LANES):
                rows0[i, pl.ds(j * LANES, LANES)] = jnp.zeros((LANES,), jnp.float32)

        off = 0
        for blk in zsizes:
            pltpu.sync_copy(rows0.at[pl.ds(0, blk)],
                            accum.at[pl.ds(s * zrows + off, blk)])
            off += blk
        if tail:
            @pl.when(s == NS - 1)
            def _zero_tail():
                pltpu.sync_copy(rows0.at[pl.ds(0, tail)],
                                accum.at[pl.ds(NS * zrows, tail)])

        cbase = (c * NS + s) * cpw

        pltpu.sync_copy(pk3_hbm.at[cbase], tb0)
        pltpu.async_copy(x_hbm.at[tb0.at[1]], rows0, gsem)
        pltpu.async_copy(pk3_hbm.at[cbase + 1], tb1, isem1)

        plsc.subcore_barrier()

        @pl.loop(0, cpw, step=6)
        def _edge_chunks(k):
            for p in range(6):
                i = k + p
                tb, rows = tbs[p % 3], rws[p % 2]
                tbn, rowsq = tbs[(p + 1) % 3], rws[(p + 1) % 2]
                tbf = tbs[(p + 2) % 3]

                pltpu.make_async_copy(x_hbm.at[tb.at[1]], rows, gsem).wait()

                @pl.when(i > 0)
                def _drain_scatter():
                    pltpu.make_async_copy(rowsq, accum.at[tbf.at[0]], ssem).wait()

                @pl.when(i + 1 < cpw)
                def _launch_gather():
                    pltpu.make_async_copy(pk3_hbm.at[cbase + i + 1], tbn,
                                          isems[(p + 1) % 2]).wait()
                    pltpu.async_copy(x_hbm.at[tbn.at[1]], rowsq, gsem)

                @pl.when(i + 2 < cpw)
                def _prefetch_idx():
                    pltpu.async_copy(pk3_hbm.at[cbase + i + 2], tbf,
                                     isems[p % 2])

                @pl.loop(0, CHUNK, step=LANES)
                def _gate_scale(g):
                    idxs = tb[0, pl.ds(g, LANES)]
                    idxd = tb[1, pl.ds(g, LANES)]
                    s1 = plsc.load_gather(x1t, [idxs])
                    s2 = plsc.load_gather(x2t, [idxd])
                    av = plsc.bitcast(tb[2, pl.ds(g, LANES)], jnp.float32)
                    e2 = jnp.exp((s1 + s2) * 2.0)
                    mv = (1.0 - 2.0 / (e2 + 1.0)) * av
                    for ii in range(LANES):
                        mi = mv[ii]
                        for j in range(d // LANES):
                            sl = pl.ds(j * LANES, LANES)
                            rows[g + ii, sl] = rows[g + ii, sl] * mi

                pltpu.async_copy(rows, accum.at[tb.at[0]], ssem, add=True)

        pltpu.make_async_copy(rows1, accum.at[tb2.at[0]], ssem).wait()

        plsc.subcore_barrier()

        r0 = s * zrows
        pltpu.sync_copy(accum.at[pl.ds(r0, zrows)], out_hbm.at[c, pl.ds(r0, zrows)])
        if tail:
            @pl.when(s == NS - 1)
            def _write_tail():
                pltpu.sync_copy(accum.at[pl.ds(NS * zrows, tail)],
                                out_hbm.at[c, pl.ds(NS * zrows, tail)])

    return sc_kernel(x, pk3, x1, x2)


def kernel(x, edge_index, adj_values, W1, W2):
    e = edge_index.shape[1]

    x1, x2 = _gates(x, W1, W2)

    quantum = NC * NS * CHUNK * 6
    epad = ((e + quantum - 1) // quantum) * quantum
    pad = epad - e
    src = jnp.concatenate([edge_index[0], jnp.zeros((pad,), jnp.int32)])
    dst = jnp.concatenate([edge_index[1], jnp.zeros((pad,), jnp.int32)])
    adj = jnp.concatenate([adj_values, jnp.zeros((pad,), jnp.float32)])
    nctot = epad // CHUNK
    pk3 = jnp.stack(
        [src.reshape(nctot, CHUNK),
         dst.reshape(nctot, CHUNK),
         lax.bitcast_convert_type(adj, jnp.int32).reshape(nctot, CHUNK)],
        axis=1)

    partials = _sc_aggregate(x, pk3, x1, x2)
    return _sum_partials(partials)
